# baseline (speedup 1.0000x reference)
import functools
import math

import jax
import jax.numpy as jnp
from jax import lax
from jax.experimental import pallas as pl
from jax.experimental.pallas import tpu as pltpu
from jax.experimental.pallas import tpu_sc as plsc

M = 1048576
N = 262144
DIST_TH2 = 0.05 * 0.05
DOT_TH = math.cos(20.0 * math.pi / 180.0)
DOT_TH2 = DOT_TH * DOT_TH
SIGMA = 0.6
INV_2SIG2 = 1.0 / (2.0 * SIGMA * SIGMA)

NC = 2
NS = 16
NW = NC * NS
NPW = N // NW
CH = 2048
NGC = CH // 16
RPW = M // NW
CHP = 2048
NGP = CHP // 16

_SC_PARAMS = pltpu.CompilerParams(
    needs_layout_passes=False, use_tc_tiling_on_sc=False)


def _sc_find_fuse(mcols, mconf, fcols, idx):
  mesh = plsc.VectorSubcoreMesh(core_axis_name="c", subcore_axis_name="s")

  @functools.partial(
      pl.kernel,
      out_type=(
          jax.ShapeDtypeStruct((NW, 16), jnp.int32),
          jax.ShapeDtypeStruct((N,), jnp.int32),
          jax.ShapeDtypeStruct((N, 16), jnp.float32),
      ),
      mesh=mesh,
      compiler_params=_SC_PARAMS,
      scratch_types=[
          pltpu.VMEM((CH,), jnp.int32),
          pltpu.VMEM((CH,), jnp.float32),
          pltpu.VMEM((CH,), jnp.float32),
          pltpu.VMEM((CH,), jnp.float32),
          pltpu.VMEM((CH,), jnp.float32),
          pltpu.VMEM((CH,), jnp.float32),
          pltpu.VMEM((CH,), jnp.float32),
          pltpu.VMEM((CH,), jnp.float32),
          pltpu.VMEM((CH,), jnp.float32),
          pltpu.VMEM((CH,), jnp.float32),
          pltpu.VMEM((CH,), jnp.float32),
          pltpu.VMEM((CH,), jnp.float32),
          pltpu.VMEM((CH,), jnp.float32),
          pltpu.VMEM((NPW + 16,), jnp.int32),
          pltpu.VMEM((NPW + 16,), jnp.int32),
          pltpu.VMEM((16,), jnp.int32),
          pltpu.VMEM((16,), jnp.int32),
          pltpu.VMEM((16,), jnp.int32),
          pltpu.VMEM((16,), jnp.float32),
          pltpu.VMEM((16,), jnp.float32),
          pltpu.VMEM((16,), jnp.float32),
          pltpu.VMEM((16,), jnp.float32),
          pltpu.VMEM((16, 16), jnp.float32),
          pltpu.SemaphoreType.DMA,
      ],
  )
  def k(mpx_h, mpy_h, mpz_h, mnx_h, mny_h, mnz_h, mcx_h, mcy_h, mcz_h,
        mconf_h,
        fpx_h, fpy_h, fpz_h, fnx_h, fny_h, fnz_h, fcx_h, fcy_h, fcz_h,
        idx_h,
        counts_h, eidx_h, erow_h,
        idx_v, fpx_v, fpy_v, fpz_v, fnx_v, fny_v, fnz_v,
        mpx_v, mpy_v, mpz_v, mnx_v, mny_v, mnz_v,
        gi_list, idx_list, cnt_v, gi16, idx16,
        ax_s, ay_s, az_s, conf16, row16, sem):
    wid = lax.axis_index("s") * NC + lax.axis_index("c")
    base = wid * NPW
    iot = lax.iota(jnp.int32, 16)

    kcount = jnp.int32(0)
    for half in range(NPW // CH):
      cbase = base + half * CH
      sl_h = pl.ds(cbase, CH)

      d0 = pltpu.async_copy(idx_h.at[sl_h], idx_v, sem)
      dfs = [pltpu.async_copy(src.at[sl_h], dst, sem)
             for src, dst in ((fpx_h, fpx_v), (fpy_h, fpy_v),
                              (fpz_h, fpz_v), (fnx_h, fnx_v),
                              (fny_h, fny_v), (fnz_h, fnz_v))]
      d0.wait()
      dgs = [pltpu.async_copy(src.at[idx_v], dst, sem)
             for src, dst in ((mpx_h, mpx_v), (mpy_h, mpy_v),
                              (mpz_h, mpz_v), (mnx_h, mnx_v),
                              (mny_h, mny_v), (mnz_h, mnz_v))]
      for d in dfs:
        d.wait()
      for d in dgs:
        d.wait()

      def group(g, ptr, cbase=cbase):
        sl = pl.ds(g * 16, 16)
        dx = fpx_v[sl] - mpx_v[sl]
        dy = fpy_v[sl] - mpy_v[sl]
        dz = fpz_v[sl] - mpz_v[sl]
        d2_ = dx * dx + dy * dy + dz * dz
        fnx = fnx_v[sl]
        fny = fny_v[sl]
        fnz = fnz_v[sl]
        mnx = mnx_v[sl]
        mny = mny_v[sl]
        mnz = mnz_v[sl]
        dot = mnx * fnx + mny * fny + mnz * fnz
        nm2 = mnx * mnx + mny * mny + mnz * mnz
        nf2 = fnx * fnx + fny * fny + fnz * fnz
        valid = ((d2_ < DIST_TH2) & (dot > 0.0)
                 & (dot * dot > DOT_TH2 * nm2 * nf2))
        cnt = jnp.max(plsc.all_reduce_population_count(valid))

        @pl.when(cnt > 0)
        def _():
          gi_vals = cbase + g * 16 + iot
          idx16v = idx_v[sl]
          plsc.store_compressed(gi_list.at[pl.ds(ptr, 16)], gi_vals,
                                mask=valid)
          plsc.store_compressed(idx_list.at[pl.ds(ptr, 16)], idx16v,
                                mask=valid)

        return ptr + cnt

      kcount = lax.fori_loop(0, NGC, group, kcount)

    nfix = (kcount + 15) // 16
    cnt_v[...] = jnp.full((16,), nfix * 16, jnp.int32)
    pltpu.sync_copy(cnt_v, counts_h.at[wid])

    zeros16 = jnp.zeros((16,), jnp.int32)
    gi_list[pl.ds(kcount, 16)] = zeros16
    idx_list[pl.ds(kcount, 16)] = zeros16

    def gather3(xh, yh, zh, iref):
      h1 = pltpu.async_copy(xh.at[iref], ax_s, sem)
      h2 = pltpu.async_copy(yh.at[iref], ay_s, sem)
      h3 = pltpu.async_copy(zh.at[iref], az_s, sem)
      h1.wait()
      h2.wait()
      h3.wait()
      return ax_s[...], ay_s[...], az_s[...]

    def fix(t, _):
      e = t * 16
      gi16[...] = gi_list[pl.ds(e, 16)]
      idx16[...] = idx_list[pl.ds(e, 16)]

      fpx, fpy, fpz = gather3(fpx_h, fpy_h, fpz_h, gi16)
      r2 = fpx * fpx + fpy * fpy + fpz * fpz
      dc2 = (fpx * fpx + fpy * fpy) / (r2 + 1e-30)
      alpha = jnp.exp(-dc2 * INV_2SIG2)

      hc = pltpu.async_copy(mconf_h.at[idx16], conf16, sem)
      hc.wait()
      w = conf16[...]

      real = (e + iot) < kcount
      inv_den = 1.0 / (w + alpha)
      wd = jnp.where(real, w * inv_den, 1.0)
      ad = jnp.where(real, alpha * inv_den, 0.0)

      c9 = jnp.full((16,), 9, jnp.int32)
      plsc.store_scatter(row16, [iot, c9], jnp.where(real, w + alpha, w))
      fps = (fpx, fpy, fpz)
      mps = gather3(mpx_h, mpy_h, mpz_h, idx16)
      for c in range(3):
        cc = jnp.full((16,), c, jnp.int32)
        plsc.store_scatter(row16, [iot, cc], wd * mps[c] + ad * fps[c])
      fns = gather3(fnx_h, fny_h, fnz_h, gi16)
      mns = gather3(mnx_h, mny_h, mnz_h, idx16)
      for c in range(3):
        cc = jnp.full((16,), 3 + c, jnp.int32)
        plsc.store_scatter(row16, [iot, cc], wd * mns[c] + ad * fns[c])
      fcs = gather3(fcx_h, fcy_h, fcz_h, gi16)
      mcs = gather3(mcx_h, mcy_h, mcz_h, idx16)
      for c in range(3):
        cc = jnp.full((16,), 6 + c, jnp.int32)
        plsc.store_scatter(row16, [iot, cc], wd * mcs[c] + ad * fcs[c])

      pltpu.sync_copy(row16, erow_h.at[pl.ds(base + e, 16)])
      pltpu.sync_copy(idx16, eidx_h.at[pl.ds(base + e, 16)])
      return 0

    lax.fori_loop(0, nfix, fix, 0)

  return k(*mcols, mconf, *fcols, idx)


def _sc_pack_apply(mcols, mconf, counts, eidx, erow):
  mesh = plsc.VectorSubcoreMesh(core_axis_name="c", subcore_axis_name="s")

  @functools.partial(
      pl.kernel,
      out_type=jax.ShapeDtypeStruct((M, 16), jnp.float32),
      mesh=mesh,
      compiler_params=_SC_PARAMS,
      scratch_types=[
          [pltpu.VMEM((CHP,), jnp.float32)] * 9,
          pltpu.VMEM((CHP,), jnp.float32),
          pltpu.VMEM((CHP, 16), jnp.float32),
          pltpu.VMEM((NW, 16), jnp.int32),
          pltpu.VMEM((16,), jnp.int32),
          pltpu.VMEM((16, 16), jnp.float32),
          pltpu.SemaphoreType.DMA,
      ],
  )
  def k(mpx_h, mpy_h, mpz_h, mnx_h, mny_h, mnz_h, mcx_h, mcy_h, mcz_h,
        mconf_h, counts_h, eidx_h, erow_h, out_h,
        cols_v, cf_pv, ob_v, counts_v, tgt16, rb_v, sem):
    wid = lax.axis_index("s") * NC + lax.axis_index("c")
    rbase = wid * RPW
    iot = lax.iota(jnp.int32, 16)
    dcnt = pltpu.async_copy(counts_h, counts_v, sem)
    srcs = (mpx_h, mpy_h, mpz_h, mnx_h, mny_h, mnz_h, mcx_h, mcy_h, mcz_h)

    for chunk in range(RPW // CHP):
      r0 = rbase + chunk * CHP
      sl_h = pl.ds(r0, CHP)
      ds_ = [pltpu.async_copy(src.at[sl_h], dst, sem)
             for src, dst in zip(srcs, cols_v)]
      ds_.append(pltpu.async_copy(mconf_h.at[sl_h], cf_pv, sem))
      for d in ds_:
        d.wait()

      def pgroup(g, _):
        rows = g * 16 + iot
        sl = pl.ds(g * 16, 16)
        for c in range(9):
          cc = jnp.full((16,), c, jnp.int32)
          plsc.store_scatter(ob_v, [rows, cc], cols_v[c][sl])
        c9 = jnp.full((16,), 9, jnp.int32)
        plsc.store_scatter(ob_v, [rows, c9], cf_pv[sl])
        return 0

      lax.fori_loop(0, NGP, pgroup, 0)
      pltpu.sync_copy(ob_v, out_h.at[pl.ds(r0, CHP)])

    dcnt.wait()
    for wsrc in range(NW):
      cnt = jnp.max(counts_v[wsrc])

      def abody(t, _, wsrc=wsrc):
        j = wsrc * NPW + t * 16
        pltpu.sync_copy(eidx_h.at[pl.ds(j, 16)], tgt16)
        pltpu.sync_copy(erow_h.at[pl.ds(j, 16)], rb_v)
        pltpu.async_copy(rb_v, out_h.at[tgt16], sem).wait()
        return 0

      lax.fori_loop(0, cnt // 16, abody, 0)

  return k(*mcols, mconf, counts, eidx, erow)


def _soa(a):
  return (a[:, 0], a[:, 1], a[:, 2])


def kernel(map_points, map_normals, map_colors, map_confidence,
           frame_points, frame_normals, frame_colors, idx):
  mcols = (*_soa(map_points), *_soa(map_normals), *_soa(map_colors))
  fcols = (*_soa(frame_points), *_soa(frame_normals), *_soa(frame_colors))
  counts, eidx, erow = _sc_find_fuse(mcols, map_confidence, fcols, idx)
  return _sc_pack_apply(mcols, map_confidence, counts, eidx, erow)[:, :10]


# speedup vs baseline: 1.0174x; 1.0174x over previous
import functools
import math

import jax
import jax.numpy as jnp
from jax import lax
from jax.experimental import pallas as pl
from jax.experimental.pallas import tpu as pltpu
from jax.experimental.pallas import tpu_sc as plsc

M = 1048576
N = 262144
DIST_TH2 = 0.05 * 0.05
DOT_TH = math.cos(20.0 * math.pi / 180.0)
DOT_TH2 = DOT_TH * DOT_TH
SIGMA = 0.6
INV_2SIG2 = 1.0 / (2.0 * SIGMA * SIGMA)

NC = 2
NS = 16
NW = NC * NS
NPW = N // NW
CH = 4096
NGC = CH // 16
RPW = M // NW
CHP = 4096
NGP = CHP // 16

_SC_PARAMS = pltpu.CompilerParams(
    needs_layout_passes=False, use_tc_tiling_on_sc=False)


def _sc_find_fuse(mcols, mconf, fcols, idx):
  mesh = plsc.VectorSubcoreMesh(core_axis_name="c", subcore_axis_name="s")

  @functools.partial(
      pl.kernel,
      out_type=(
          jax.ShapeDtypeStruct((NW, 16), jnp.int32),
          jax.ShapeDtypeStruct((N,), jnp.int32),
          jax.ShapeDtypeStruct((N, 16), jnp.float32),
      ),
      mesh=mesh,
      compiler_params=_SC_PARAMS,
      scratch_types=[
          pltpu.VMEM((CH,), jnp.int32),
          pltpu.VMEM((CH,), jnp.float32),
          pltpu.VMEM((CH,), jnp.float32),
          pltpu.VMEM((CH,), jnp.float32),
          pltpu.VMEM((CH,), jnp.float32),
          pltpu.VMEM((CH,), jnp.float32),
          pltpu.VMEM((CH,), jnp.float32),
          pltpu.VMEM((CH,), jnp.float32),
          pltpu.VMEM((CH,), jnp.float32),
          pltpu.VMEM((CH,), jnp.float32),
          pltpu.VMEM((CH,), jnp.float32),
          pltpu.VMEM((CH,), jnp.float32),
          pltpu.VMEM((CH,), jnp.float32),
          pltpu.VMEM((NPW + 16,), jnp.int32),
          pltpu.VMEM((NPW + 16,), jnp.int32),
          pltpu.VMEM((16,), jnp.int32),
          pltpu.VMEM((16,), jnp.int32),
          pltpu.VMEM((16,), jnp.int32),
          pltpu.VMEM((16,), jnp.float32),
          pltpu.VMEM((16,), jnp.float32),
          pltpu.VMEM((16,), jnp.float32),
          pltpu.VMEM((16,), jnp.float32),
          pltpu.VMEM((16, 16), jnp.float32),
          pltpu.SemaphoreType.DMA,
      ],
  )
  def k(mpx_h, mpy_h, mpz_h, mnx_h, mny_h, mnz_h, mcx_h, mcy_h, mcz_h,
        mconf_h,
        fpx_h, fpy_h, fpz_h, fnx_h, fny_h, fnz_h, fcx_h, fcy_h, fcz_h,
        idx_h,
        counts_h, eidx_h, erow_h,
        idx_v, fpx_v, fpy_v, fpz_v, fnx_v, fny_v, fnz_v,
        mpx_v, mpy_v, mpz_v, mnx_v, mny_v, mnz_v,
        gi_list, idx_list, cnt_v, gi16, idx16,
        ax_s, ay_s, az_s, conf16, row16, sem):
    wid = lax.axis_index("s") * NC + lax.axis_index("c")
    base = wid * NPW
    iot = lax.iota(jnp.int32, 16)

    kcount = jnp.int32(0)
    for half in range(NPW // CH):
      cbase = base + half * CH
      sl_h = pl.ds(cbase, CH)

      d0 = pltpu.async_copy(idx_h.at[sl_h], idx_v, sem)
      dfs = [pltpu.async_copy(src.at[sl_h], dst, sem)
             for src, dst in ((fpx_h, fpx_v), (fpy_h, fpy_v),
                              (fpz_h, fpz_v), (fnx_h, fnx_v),
                              (fny_h, fny_v), (fnz_h, fnz_v))]
      d0.wait()
      dgs = [pltpu.async_copy(src.at[idx_v], dst, sem)
             for src, dst in ((mpx_h, mpx_v), (mpy_h, mpy_v),
                              (mpz_h, mpz_v), (mnx_h, mnx_v),
                              (mny_h, mny_v), (mnz_h, mnz_v))]
      for d in dfs:
        d.wait()
      for d in dgs:
        d.wait()

      def group(g, ptr, cbase=cbase):
        sl = pl.ds(g * 16, 16)
        dx = fpx_v[sl] - mpx_v[sl]
        dy = fpy_v[sl] - mpy_v[sl]
        dz = fpz_v[sl] - mpz_v[sl]
        d2_ = dx * dx + dy * dy + dz * dz
        fnx = fnx_v[sl]
        fny = fny_v[sl]
        fnz = fnz_v[sl]
        mnx = mnx_v[sl]
        mny = mny_v[sl]
        mnz = mnz_v[sl]
        dot = mnx * fnx + mny * fny + mnz * fnz
        nm2 = mnx * mnx + mny * mny + mnz * mnz
        nf2 = fnx * fnx + fny * fny + fnz * fnz
        valid = ((d2_ < DIST_TH2) & (dot > 0.0)
                 & (dot * dot > DOT_TH2 * nm2 * nf2))
        cnt = jnp.max(plsc.all_reduce_population_count(valid))

        @pl.when(cnt > 0)
        def _():
          gi_vals = cbase + g * 16 + iot
          idx16v = idx_v[sl]
          plsc.store_compressed(gi_list.at[pl.ds(ptr, 16)], gi_vals,
                                mask=valid)
          plsc.store_compressed(idx_list.at[pl.ds(ptr, 16)], idx16v,
                                mask=valid)

        return ptr + cnt

      kcount = lax.fori_loop(0, NGC, group, kcount)

    nfix = (kcount + 15) // 16
    cnt_v[...] = jnp.full((16,), nfix * 16, jnp.int32)
    pltpu.sync_copy(cnt_v, counts_h.at[wid])

    zeros16 = jnp.zeros((16,), jnp.int32)
    gi_list[pl.ds(kcount, 16)] = zeros16
    idx_list[pl.ds(kcount, 16)] = zeros16

    def gather3(xh, yh, zh, iref):
      h1 = pltpu.async_copy(xh.at[iref], ax_s, sem)
      h2 = pltpu.async_copy(yh.at[iref], ay_s, sem)
      h3 = pltpu.async_copy(zh.at[iref], az_s, sem)
      h1.wait()
      h2.wait()
      h3.wait()
      return ax_s[...], ay_s[...], az_s[...]

    def fix(t, _):
      e = t * 16
      gi16[...] = gi_list[pl.ds(e, 16)]
      idx16[...] = idx_list[pl.ds(e, 16)]

      fpx, fpy, fpz = gather3(fpx_h, fpy_h, fpz_h, gi16)
      r2 = fpx * fpx + fpy * fpy + fpz * fpz
      dc2 = (fpx * fpx + fpy * fpy) / (r2 + 1e-30)
      alpha = jnp.exp(-dc2 * INV_2SIG2)

      hc = pltpu.async_copy(mconf_h.at[idx16], conf16, sem)
      hc.wait()
      w = conf16[...]

      real = (e + iot) < kcount
      inv_den = 1.0 / (w + alpha)
      wd = jnp.where(real, w * inv_den, 1.0)
      ad = jnp.where(real, alpha * inv_den, 0.0)

      c9 = jnp.full((16,), 9, jnp.int32)
      plsc.store_scatter(row16, [iot, c9], jnp.where(real, w + alpha, w))
      fps = (fpx, fpy, fpz)
      mps = gather3(mpx_h, mpy_h, mpz_h, idx16)
      for c in range(3):
        cc = jnp.full((16,), c, jnp.int32)
        plsc.store_scatter(row16, [iot, cc], wd * mps[c] + ad * fps[c])
      fns = gather3(fnx_h, fny_h, fnz_h, gi16)
      mns = gather3(mnx_h, mny_h, mnz_h, idx16)
      for c in range(3):
        cc = jnp.full((16,), 3 + c, jnp.int32)
        plsc.store_scatter(row16, [iot, cc], wd * mns[c] + ad * fns[c])
      fcs = gather3(fcx_h, fcy_h, fcz_h, gi16)
      mcs = gather3(mcx_h, mcy_h, mcz_h, idx16)
      for c in range(3):
        cc = jnp.full((16,), 6 + c, jnp.int32)
        plsc.store_scatter(row16, [iot, cc], wd * mcs[c] + ad * fcs[c])

      pltpu.sync_copy(row16, erow_h.at[pl.ds(base + e, 16)])
      pltpu.sync_copy(idx16, eidx_h.at[pl.ds(base + e, 16)])
      return 0

    lax.fori_loop(0, nfix, fix, 0)

  return k(*mcols, mconf, *fcols, idx)


def _sc_pack_apply(mcols, mconf, counts, eidx, erow):
  mesh = plsc.VectorSubcoreMesh(core_axis_name="c", subcore_axis_name="s")

  @functools.partial(
      pl.kernel,
      out_type=jax.ShapeDtypeStruct((M, 16), jnp.float32),
      mesh=mesh,
      compiler_params=_SC_PARAMS,
      scratch_types=[
          [pltpu.VMEM((CHP,), jnp.float32)] * 9,
          pltpu.VMEM((CHP,), jnp.float32),
          pltpu.VMEM((CHP, 16), jnp.float32),
          pltpu.VMEM((NW, 16), jnp.int32),
          pltpu.VMEM((16,), jnp.int32),
          pltpu.VMEM((16, 16), jnp.float32),
          pltpu.SemaphoreType.DMA,
      ],
  )
  def k(mpx_h, mpy_h, mpz_h, mnx_h, mny_h, mnz_h, mcx_h, mcy_h, mcz_h,
        mconf_h, counts_h, eidx_h, erow_h, out_h,
        cols_v, cf_pv, ob_v, counts_v, tgt16, rb_v, sem):
    wid = lax.axis_index("s") * NC + lax.axis_index("c")
    rbase = wid * RPW
    iot = lax.iota(jnp.int32, 16)
    dcnt = pltpu.async_copy(counts_h, counts_v, sem)
    srcs = (mpx_h, mpy_h, mpz_h, mnx_h, mny_h, mnz_h, mcx_h, mcy_h, mcz_h)

    for chunk in range(RPW // CHP):
      r0 = rbase + chunk * CHP
      sl_h = pl.ds(r0, CHP)
      ds_ = [pltpu.async_copy(src.at[sl_h], dst, sem)
             for src, dst in zip(srcs, cols_v)]
      ds_.append(pltpu.async_copy(mconf_h.at[sl_h], cf_pv, sem))
      for d in ds_:
        d.wait()

      def pgroup(g, _):
        rows = g * 16 + iot
        sl = pl.ds(g * 16, 16)
        for c in range(9):
          cc = jnp.full((16,), c, jnp.int32)
          plsc.store_scatter(ob_v, [rows, cc], cols_v[c][sl])
        c9 = jnp.full((16,), 9, jnp.int32)
        plsc.store_scatter(ob_v, [rows, c9], cf_pv[sl])
        return 0

      lax.fori_loop(0, NGP, pgroup, 0)
      pltpu.sync_copy(ob_v, out_h.at[pl.ds(r0, CHP)])

    dcnt.wait()
    for wsrc in range(NW):
      cnt = jnp.max(counts_v[wsrc])

      def abody(t, _, wsrc=wsrc):
        j = wsrc * NPW + t * 16
        pltpu.sync_copy(eidx_h.at[pl.ds(j, 16)], tgt16)
        pltpu.sync_copy(erow_h.at[pl.ds(j, 16)], rb_v)
        pltpu.async_copy(rb_v, out_h.at[tgt16], sem).wait()
        return 0

      lax.fori_loop(0, cnt // 16, abody, 0)

  return k(*mcols, mconf, counts, eidx, erow)


def _soa(a):
  return (a[:, 0], a[:, 1], a[:, 2])


def kernel(map_points, map_normals, map_colors, map_confidence,
           frame_points, frame_normals, frame_colors, idx):
  mcols = (*_soa(map_points), *_soa(map_normals), *_soa(map_colors))
  fcols = (*_soa(frame_points), *_soa(frame_normals), *_soa(frame_colors))
  counts, eidx, erow = _sc_find_fuse(mcols, map_confidence, fcols, idx)
  return _sc_pack_apply(mcols, map_confidence, counts, eidx, erow)[:, :10]
